# baseline (speedup 1.0000x reference)
import functools

import jax
import jax.numpy as jnp
from jax import lax
from jax.experimental import pallas as pl
from jax.experimental.pallas import tpu as pltpu
from jax.experimental.pallas import tpu_sc as plsc

D = 128
ACC_W = D + 16
NC = 2
NS = 16
NW = NC * NS
CH = 80
_EPS = 1e-5
_PREC = jax.lax.Precision.DEFAULT
_SC_PARAMS = pltpu.CompilerParams(use_tc_tiling_on_sc=False)


def _dot_t(x, w):
    return lax.dot_general(x, w, (((1,), (1,)), ((), ())),
                           preferred_element_type=jnp.float32, precision=_PREC)


def _tables_phase1(h_nodes, h_nets, w1a1, w1b1, b11, w1b2, b12,
                   block_rows=1000):
    n = h_nodes.shape[0]

    def body(hn_ref, hv_ref, wa_ref, wb1_ref, b11_ref, wb2_ref, b12_ref,
             a1_ref, b1_ref, b2_ref):
        hn = hn_ref[...]
        a1_ref[...] = _dot_t(hn, wa_ref[...])
        b1_ref[...] = _dot_t(hv_ref[...], wb1_ref[...]) + b11_ref[...]
        b2_ref[...] = _dot_t(hn, wb2_ref[...]) + b12_ref[...]

    rows = lambda i: (i, 0)
    full = lambda i: (0, 0)
    out = jax.ShapeDtypeStruct((n, D), jnp.float32)
    return pl.pallas_call(
        body,
        grid=(n // block_rows,),
        in_specs=[
            pl.BlockSpec((block_rows, D), rows),
            pl.BlockSpec((block_rows, D), rows),
            pl.BlockSpec((D, D), full),
            pl.BlockSpec((D, D), full),
            pl.BlockSpec((1, D), full),
            pl.BlockSpec((D, D), full),
            pl.BlockSpec((1, D), full),
        ],
        out_specs=[pl.BlockSpec((block_rows, D), rows)] * 3,
        out_shape=[out, out, out],
    )(h_nodes, h_nets, w1a1, w1b1, b11.reshape(1, D), w1b2, b12.reshape(1, D))


def _sc_gather_add(a_tab, b_tab, src, dst):
    e_total = src.shape[0]
    per_w = e_total // NW
    assert e_total % NW == 0 and per_w % 8 == 0
    for chg in range(128, 7, -8):
        nfull = per_w // chg
        tail = per_w - nfull * chg
        if nfull >= 6 and nfull % 2 == 0 and tail > 0:
            break
    assert nfull % 2 == 0 and nfull >= 6 and 0 < tail <= chg and tail % 8 == 0
    mesh = plsc.VectorSubcoreMesh(core_axis_name="c", subcore_axis_name="s",
                                  num_cores=NC, num_subcores=NS)

    @functools.partial(
        pl.kernel,
        out_type=jax.ShapeDtypeStruct((e_total, D), jnp.float32),
        mesh=mesh,
        scratch_types=[
            pltpu.VMEM((per_w,), jnp.int32),
            pltpu.VMEM((per_w,), jnp.int32),
            pltpu.VMEM((chg, D), jnp.float32),
            pltpu.VMEM((chg, D), jnp.float32),
            pltpu.VMEM((chg, D), jnp.float32),
            pltpu.VMEM((chg, D), jnp.float32),
            pltpu.SemaphoreType.DMA,
            pltpu.SemaphoreType.DMA,
            pltpu.SemaphoreType.DMA,
            pltpu.SemaphoreType.DMA,
            pltpu.SemaphoreType.DMA,
            pltpu.SemaphoreType.DMA,
        ],
        compiler_params=_SC_PARAMS,
    )
    def k(a_hbm, b_hbm, src_hbm, dst_hbm, out_hbm,
          sidx_all, didx_all, bufa0, bufa1, bufb0, bufb1,
          sema0, sema1, semb0, semb1, semw0, semw1):
        wid = lax.axis_index("s") * NC + lax.axis_index("c")
        base = wid * per_w
        slots = ((bufa0, bufb0, sema0, semb0, semw0),
                 (bufa1, bufb1, sema1, semb1, semw1))

        pltpu.sync_copy(src_hbm.at[pl.ds(base, per_w)], sidx_all)
        pltpu.sync_copy(dst_hbm.at[pl.ds(base, per_w)], didx_all)

        def bufs(s, sz):
            bufa, bufb, _, _, _ = slots[s]
            if sz == chg:
                return bufa, bufb
            return bufa.at[pl.ds(0, sz)], bufb.at[pl.ds(0, sz)]

        def fire_gather(c, s, sz=chg):
            bufa, bufb = bufs(s, sz)
            _, _, sema, semb, _ = slots[s]
            off = c * chg
            pltpu.async_copy(a_hbm.at[sidx_all.at[pl.ds(off, sz)]], bufa, sema)
            pltpu.async_copy(b_hbm.at[didx_all.at[pl.ds(off, sz)]], bufb, semb)

        def wait_gather(c, s, sz=chg):
            bufa, bufb = bufs(s, sz)
            _, _, sema, semb, _ = slots[s]
            off = c * chg
            pltpu.make_async_copy(a_hbm.at[sidx_all.at[pl.ds(off, sz)]], bufa,
                                  sema).wait()
            pltpu.make_async_copy(b_hbm.at[didx_all.at[pl.ds(off, sz)]], bufb,
                                  semb).wait()

        def add_slot(s, sz=chg):
            bufa, bufb, _, _, _ = slots[s]

            def row(i, cc):
                for j in range(D // 16):
                    sl = pl.ds(j * 16, 16)
                    plsc.addupdate(bufa.at[i, sl], bufb[i, sl])
                return cc

            lax.fori_loop(0, sz, row, 0)

        def fire_wb(c, s, sz=chg):
            bufa, _ = bufs(s, sz)
            _, _, _, _, semw = slots[s]
            pltpu.async_copy(bufa, out_hbm.at[pl.ds(base + c * chg, sz)], semw)

        def wait_wb(c, s, sz=chg):
            bufa, _ = bufs(s, sz)
            _, _, _, _, semw = slots[s]
            pltpu.make_async_copy(bufa, out_hbm.at[pl.ds(base + c * chg, sz)],
                                  semw).wait()

        def step(c, s, *, wb_wait=True, fire1=True, next_sz=chg, sz=chg):
            wait_gather(c, s, sz)
            if fire1:
                if wb_wait:
                    wait_wb(c - 1, s ^ 1)
                fire_gather(c + 1, s ^ 1, next_sz)
            add_slot(s, sz)
            fire_wb(c, s, sz)

        fire_gather(0, 0)
        step(0, 0, wb_wait=False)

        def pair(r, cc):
            c = 2 * r + 1
            step(c, 1)
            step(c + 1, 0)
            return cc

        lax.fori_loop(0, (nfull - 2) // 2, pair, 0)
        step(nfull - 1, 1, next_sz=tail)
        step(nfull, 0, fire1=False, sz=tail)
        wait_wb(nfull - 1, 1)
        wait_wb(nfull, 0, tail)

    return k(a_tab, b_tab, src, dst)


def _edge_mlp(g, attr, w1c, g1, bt1, w2, b2, block_rows=2000):
    e, _ = g.shape
    de = attr.shape[1]

    def body(g_ref, a_ref, w1c_ref, g1_ref, bt1_ref, w2_ref, b2_ref, o_ref):
        x = g_ref[...] + _dot_t(a_ref[...], w1c_ref[...])
        mu = jnp.mean(x, axis=-1, keepdims=True)
        var = jnp.mean((x - mu) ** 2, axis=-1, keepdims=True)
        h = (x - mu) * lax.rsqrt(var + _EPS) * g1_ref[...] + bt1_ref[...]
        h = jnp.maximum(h, 0.0)
        o_ref[...] = _dot_t(h, w2_ref[...]) + b2_ref[...]

    return pl.pallas_call(
        body,
        grid=(e // block_rows,),
        in_specs=[
            pl.BlockSpec((block_rows, D), lambda i: (i, 0)),
            pl.BlockSpec((block_rows, de), lambda i: (i, 0)),
            pl.BlockSpec((D, de), lambda i: (0, 0)),
            pl.BlockSpec((1, D), lambda i: (0, 0)),
            pl.BlockSpec((1, D), lambda i: (0, 0)),
            pl.BlockSpec((D, D), lambda i: (0, 0)),
            pl.BlockSpec((1, D), lambda i: (0, 0)),
        ],
        out_specs=pl.BlockSpec((block_rows, D), lambda i: (i, 0)),
        out_shape=jax.ShapeDtypeStruct((e, D), jnp.float32),
    )(g, attr, w1c, g1.reshape(1, D), bt1.reshape(1, D), w2, b2.reshape(1, D))


def _sc_scatter(msg, dst, n):
    e_total = dst.shape[0]
    per_w = e_total // NW
    assert e_total % NW == 0 and per_w % 8 == 0
    chs = 80
    nfull = per_w // chs
    tail = per_w - nfull * chs
    if tail == 0:
        nfull -= 1
        tail = chs
    nsteps = nfull + 1
    assert nsteps >= 6 and tail % 8 == 0
    stripe = n // NS
    zr = 25
    nz = stripe // zr
    mesh = plsc.VectorSubcoreMesh(core_axis_name="c", subcore_axis_name="s",
                                  num_cores=NC, num_subcores=NS)

    @functools.partial(
        pl.kernel,
        out_type=jax.ShapeDtypeStruct((NC, n, ACC_W), jnp.float32),
        mesh=mesh,
        scratch_types=[
            pltpu.VMEM((per_w,), jnp.int32),
            pltpu.VMEM((chs, ACC_W), jnp.float32),
            pltpu.VMEM((chs, ACC_W), jnp.float32),
            pltpu.VMEM((zr, ACC_W), jnp.float32),
            pltpu.VMEM_SHARED((n, ACC_W), jnp.float32),
            pltpu.SemaphoreType.DMA,
            pltpu.SemaphoreType.DMA,
            pltpu.SemaphoreType.DMA,
            pltpu.SemaphoreType.DMA,
            pltpu.SemaphoreType.DMA,
        ],
        compiler_params=_SC_PARAMS,
    )
    def k(msg_hbm, dst_hbm, out_hbm, idx_all, sbuf0, sbuf1, zbuf, acc,
          seml0, seml1, semsc0, semsc1, semz):
        cid = lax.axis_index("c")
        sid = lax.axis_index("s")
        wid = sid * NC + cid
        base = wid * per_w
        slots = ((sbuf0, seml0, semsc0), (sbuf1, seml1, semsc1))

        pltpu.sync_copy(dst_hbm.at[pl.ds(base, per_w)], idx_all)

        zero16 = jnp.zeros((16,), jnp.float32)
        one_first = jnp.where(lax.iota(jnp.int32, 16) == 0, 1.0, 0.0)

        def zrow(i, cc):
            for j in range(ACC_W // 16):
                zbuf[i, pl.ds(j * 16, 16)] = zero16
            return cc

        lax.fori_loop(0, zr, zrow, 0)

        def srow(i, cc):
            sbuf0[i, pl.ds(D, 16)] = one_first
            sbuf1[i, pl.ds(D, 16)] = one_first
            return cc

        lax.fori_loop(0, chs, srow, 0)

        def zdst(j):
            return acc.at[pl.ds(sid * stripe + j * zr, zr)]

        for j in range(nz):
            pltpu.async_copy(zbuf, zdst(j), semz)
        for j in range(nz):
            pltpu.make_async_copy(zbuf, zdst(j), semz).wait()
        plsc.subcore_barrier()

        def sz_of(c):
            return tail if c == nsteps - 1 else chs

        def sb(s, sz):
            sbuf = slots[s][0]
            return sbuf if sz == chs else sbuf.at[pl.ds(0, sz)]

        def fire_load(c, s, sz=chs):
            _, seml, _ = slots[s]
            off = base + c * chs
            pltpu.async_copy(msg_hbm.at[pl.ds(off, sz)],
                             sb(s, sz).at[:, pl.ds(0, D)], seml)

        def wait_load(c, s, sz=chs):
            _, seml, _ = slots[s]
            off = base + c * chs
            pltpu.make_async_copy(msg_hbm.at[pl.ds(off, sz)],
                                  sb(s, sz).at[:, pl.ds(0, D)], seml).wait()

        def fire_scatter(c, s, sz=chs):
            _, _, semsc = slots[s]
            idxv = idx_all.at[pl.ds(c * chs, sz)]
            pltpu.async_copy(sb(s, sz), acc.at[idxv], semsc, add=True)

        def wait_scatter(c, s, sz=chs):
            _, _, semsc = slots[s]
            idxv = idx_all.at[pl.ds(c * chs, sz)]
            pltpu.make_async_copy(sb(s, sz), acc.at[idxv], semsc).wait()

        def step(c, s, *, scat_wait=True, fire1=True, sz=chs, next_sz=chs):
            wait_load(c, s, sz)
            if fire1:
                if scat_wait:
                    wait_scatter(c - 1, s ^ 1)
                fire_load(c + 1, s ^ 1, next_sz)
            fire_scatter(c, s, sz)

        fire_load(0, 0)
        step(0, 0, scat_wait=False)
        pe = 1
        if (nsteps - 3) % 2 == 1:
            step(1, 1)
            pe = 2
        c0 = pe

        def pair(r, cc):
            c = c0 + 2 * r
            step(c, c0 % 2)
            step(c + 1, (c0 % 2) ^ 1)
            return cc

        lax.fori_loop(0, (nsteps - 2 - pe) // 2, pair, 0)
        step(nsteps - 2, (nsteps - 2) % 2, next_sz=tail)
        step(nsteps - 1, (nsteps - 1) % 2, fire1=False, sz=tail)
        wait_scatter(nsteps - 2, (nsteps - 2) % 2)
        wait_scatter(nsteps - 1, (nsteps - 1) % 2, tail)
        plsc.subcore_barrier()

        def ddst(j):
            r0 = sid * stripe + j * zr
            return (acc.at[pl.ds(r0, zr)], out_hbm.at[cid, pl.ds(r0, zr)])

        for j in range(nz):
            s_, d_ = ddst(j)
            pltpu.async_copy(s_, d_, semz)
        for j in range(nz):
            s_, d_ = ddst(j)
            pltpu.make_async_copy(s_, d_, semz).wait()

    return k(msg, dst)


def _gru_ln(psums, h_old, wih, whh, bih, bhh, g, b, w1a_next=None,
            block_rows=1000):
    n = h_old.shape[0]
    np_in = len(psums)

    def body(*refs):
        p_refs = refs[:np_in]
        h_ref, wih_ref, whh_ref, bih_ref, bhh_ref, g_ref, b_ref = \
            refs[np_in:np_in + 7]
        rest = refs[np_in + 7:]
        parts = [p_ref[i] for p_ref in p_refs for i in range(NC)]
        s = parts[0][:, :D]
        c = parts[0][:, D:D + 1]
        for p in parts[1:]:
            s = s + p[:, :D]
            c = c + p[:, D:D + 1]
        agg = s / jnp.maximum(c, 1.0)
        h = h_ref[...]
        gi = _dot_t(agg, wih_ref[...]) + bih_ref[...]
        gh = _dot_t(h, whh_ref[...]) + bhh_ref[...]
        r = jax.nn.sigmoid(gi[:, :D] + gh[:, :D])
        z = jax.nn.sigmoid(gi[:, D:2 * D] + gh[:, D:2 * D])
        nn = jnp.tanh(gi[:, 2 * D:] + r * gh[:, 2 * D:])
        hn = (1.0 - z) * nn + z * h
        mu = jnp.mean(hn, axis=-1, keepdims=True)
        var = jnp.mean((hn - mu) ** 2, axis=-1, keepdims=True)
        out = (hn - mu) * lax.rsqrt(var + _EPS) * g_ref[...] + b_ref[...]
        if w1a_next is None:
            rest[-1][...] = out
        else:
            wa_ref, o_ref, a_ref = rest
            o_ref[...] = out
            a_ref[...] = _dot_t(out, wa_ref[...])

    rows = lambda i: (i, 0)
    full = lambda i: (0, 0)
    in_specs = [
        pl.BlockSpec((NC, block_rows, ACC_W), lambda i: (0, i, 0))
        for _ in range(np_in)
    ] + [
        pl.BlockSpec((block_rows, D), rows),
        pl.BlockSpec((3 * D, D), full),
        pl.BlockSpec((3 * D, D), full),
        pl.BlockSpec((1, 3 * D), full),
        pl.BlockSpec((1, 3 * D), full),
        pl.BlockSpec((1, D), full),
        pl.BlockSpec((1, D), full),
    ]
    args = list(psums) + [h_old, wih, whh, bih.reshape(1, 3 * D),
            bhh.reshape(1, 3 * D), g.reshape(1, D), b.reshape(1, D)]
    out_spec = pl.BlockSpec((block_rows, D), rows)
    out_ty = jax.ShapeDtypeStruct((n, D), jnp.float32)
    if w1a_next is None:
        out_specs, out_shape = out_spec, out_ty
    else:
        in_specs.append(pl.BlockSpec((D, D), full))
        args.append(w1a_next)
        out_specs, out_shape = [out_spec, out_spec], [out_ty, out_ty]
    return pl.pallas_call(
        body,
        grid=(n // block_rows,),
        in_specs=in_specs,
        out_specs=out_specs,
        out_shape=out_shape,
    )(*args)


def kernel(h_nodes, h_nets, edge_index_m2n, edge_attr_m2n, edge_index_n2m,
           edge_attr_n2m, m2n_W1, m2n_b1, m2n_g1, m2n_bt1, m2n_W2, m2n_b2,
           n2m_W1, n2m_b1, n2m_g1, n2m_bt1, n2m_W2, n2m_b2,
           gru_net_Wih, gru_net_Whh, gru_net_bih, gru_net_bhh,
           gru_mac_Wih, gru_mac_Whh, gru_mac_bih, gru_mac_bhh,
           ln_net_g, ln_net_b, ln_mac_g, ln_mac_b):
    src_m = edge_index_m2n[0].astype(jnp.int32)
    dst_n = edge_index_m2n[1].astype(jnp.int32)
    src_n = edge_index_n2m[0].astype(jnp.int32)
    dst_m = edge_index_n2m[1].astype(jnp.int32)
    n_nodes = h_nodes.shape[0]
    n_nets = h_nets.shape[0]

    e_total = src_m.shape[0]
    ns = 5
    es = e_total // ns
    sl = [slice(i * es, (i + 1) * es) for i in range(ns)]

    a1, b1t, b2t = _tables_phase1(h_nodes, h_nets, m2n_W1[:, :D],
                                  m2n_W1[:, D:2 * D], m2n_b1,
                                  n2m_W1[:, D:2 * D], n2m_b1)
    g1 = [_sc_gather_add(a1, b1t, src_m[s], dst_n[s]) for s in sl]
    msg1 = [_edge_mlp(g, edge_attr_m2n[s], m2n_W1[:, 2 * D:], m2n_g1,
                      m2n_bt1, m2n_W2, m2n_b2) for g, s in zip(g1, sl)]
    psum1 = [_sc_scatter(m, dst_n[s], n_nets) for m, s in zip(msg1, sl)]
    h_nets_new, a2 = _gru_ln(tuple(psum1), h_nets, gru_net_Wih, gru_net_Whh,
                             gru_net_bih, gru_net_bhh, ln_net_g, ln_net_b,
                             w1a_next=n2m_W1[:, :D])

    g2 = [_sc_gather_add(a2, b2t, src_n[s], dst_m[s]) for s in sl]
    msg2 = [_edge_mlp(g, edge_attr_n2m[s], n2m_W1[:, 2 * D:], n2m_g1,
                      n2m_bt1, n2m_W2, n2m_b2) for g, s in zip(g2, sl)]
    psum2 = [_sc_scatter(m, dst_m[s], n_nodes) for m, s in zip(msg2, sl)]
    h_nodes_new = _gru_ln(tuple(psum2), h_nodes, gru_mac_Wih, gru_mac_Whh,
                          gru_mac_bih, gru_mac_bhh, ln_mac_g, ln_mac_b)
    return (h_nodes_new, h_nets_new)


# speedup vs baseline: 1.0734x; 1.0734x over previous
import functools

import jax
import jax.numpy as jnp
from jax import lax
from jax.experimental import pallas as pl
from jax.experimental.pallas import tpu as pltpu
from jax.experimental.pallas import tpu_sc as plsc

D = 128
ACC_W = D + 16
NC = 2
NS = 16
NW = NC * NS
CH = 80
_EPS = 1e-5
_PREC = jax.lax.Precision.DEFAULT
_SC_PARAMS = pltpu.CompilerParams(use_tc_tiling_on_sc=False)


def _dot_t(x, w):
    return lax.dot_general(x, w, (((1,), (1,)), ((), ())),
                           preferred_element_type=jnp.float32, precision=_PREC)


def _tables_phase1(h_nodes, h_nets, w1a1, w1b1, b11, w1b2, b12,
                   block_rows=1000):
    n = h_nodes.shape[0]

    def body(hn_ref, hv_ref, wa_ref, wb1_ref, b11_ref, wb2_ref, b12_ref,
             a1_ref, b1_ref, b2_ref):
        hn = hn_ref[...]
        a1_ref[...] = _dot_t(hn, wa_ref[...])
        b1_ref[...] = _dot_t(hv_ref[...], wb1_ref[...]) + b11_ref[...]
        b2_ref[...] = _dot_t(hn, wb2_ref[...]) + b12_ref[...]

    rows = lambda i: (i, 0)
    full = lambda i: (0, 0)
    out = jax.ShapeDtypeStruct((n, D), jnp.float32)
    return pl.pallas_call(
        body,
        grid=(n // block_rows,),
        in_specs=[
            pl.BlockSpec((block_rows, D), rows),
            pl.BlockSpec((block_rows, D), rows),
            pl.BlockSpec((D, D), full),
            pl.BlockSpec((D, D), full),
            pl.BlockSpec((1, D), full),
            pl.BlockSpec((D, D), full),
            pl.BlockSpec((1, D), full),
        ],
        out_specs=[pl.BlockSpec((block_rows, D), rows)] * 3,
        out_shape=[out, out, out],
    )(h_nodes, h_nets, w1a1, w1b1, b11.reshape(1, D), w1b2, b12.reshape(1, D))


def _sc_gather_add(a_tab, b_tab, src, dst):
    e_total = src.shape[0]
    per_w = e_total // NW
    assert e_total % NW == 0 and per_w % 8 == 0
    for chg in range(128, 7, -8):
        nfull = per_w // chg
        tail = per_w - nfull * chg
        if nfull >= 6 and nfull % 2 == 0 and tail > 0:
            break
    assert nfull % 2 == 0 and nfull >= 6 and 0 < tail <= chg and tail % 8 == 0
    mesh = plsc.VectorSubcoreMesh(core_axis_name="c", subcore_axis_name="s",
                                  num_cores=NC, num_subcores=NS)

    @functools.partial(
        pl.kernel,
        out_type=jax.ShapeDtypeStruct((e_total, D), jnp.float32),
        mesh=mesh,
        scratch_types=[
            pltpu.VMEM((per_w,), jnp.int32),
            pltpu.VMEM((per_w,), jnp.int32),
            pltpu.VMEM((chg, D), jnp.float32),
            pltpu.VMEM((chg, D), jnp.float32),
            pltpu.VMEM((chg, D), jnp.float32),
            pltpu.VMEM((chg, D), jnp.float32),
            pltpu.SemaphoreType.DMA,
            pltpu.SemaphoreType.DMA,
            pltpu.SemaphoreType.DMA,
            pltpu.SemaphoreType.DMA,
            pltpu.SemaphoreType.DMA,
            pltpu.SemaphoreType.DMA,
        ],
        compiler_params=_SC_PARAMS,
    )
    def k(a_hbm, b_hbm, src_hbm, dst_hbm, out_hbm,
          sidx_all, didx_all, bufa0, bufa1, bufb0, bufb1,
          sema0, sema1, semb0, semb1, semw0, semw1):
        wid = lax.axis_index("s") * NC + lax.axis_index("c")
        base = wid * per_w
        slots = ((bufa0, bufb0, sema0, semb0, semw0),
                 (bufa1, bufb1, sema1, semb1, semw1))

        pltpu.sync_copy(src_hbm.at[pl.ds(base, per_w)], sidx_all)
        pltpu.sync_copy(dst_hbm.at[pl.ds(base, per_w)], didx_all)

        def bufs(s, sz):
            bufa, bufb, _, _, _ = slots[s]
            if sz == chg:
                return bufa, bufb
            return bufa.at[pl.ds(0, sz)], bufb.at[pl.ds(0, sz)]

        def fire_gather(c, s, sz=chg):
            bufa, bufb = bufs(s, sz)
            _, _, sema, semb, _ = slots[s]
            off = c * chg
            pltpu.async_copy(a_hbm.at[sidx_all.at[pl.ds(off, sz)]], bufa, sema)
            pltpu.async_copy(b_hbm.at[didx_all.at[pl.ds(off, sz)]], bufb, semb)

        def wait_gather(c, s, sz=chg):
            bufa, bufb = bufs(s, sz)
            _, _, sema, semb, _ = slots[s]
            off = c * chg
            pltpu.make_async_copy(a_hbm.at[sidx_all.at[pl.ds(off, sz)]], bufa,
                                  sema).wait()
            pltpu.make_async_copy(b_hbm.at[didx_all.at[pl.ds(off, sz)]], bufb,
                                  semb).wait()

        def add_slot(s, sz=chg):
            bufa, bufb, _, _, _ = slots[s]

            def row(i, cc):
                for j in range(D // 16):
                    sl = pl.ds(j * 16, 16)
                    plsc.addupdate(bufa.at[i, sl], bufb[i, sl])
                return cc

            lax.fori_loop(0, sz, row, 0)

        def fire_wb(c, s, sz=chg):
            bufa, _ = bufs(s, sz)
            _, _, _, _, semw = slots[s]
            pltpu.async_copy(bufa, out_hbm.at[pl.ds(base + c * chg, sz)], semw)

        def wait_wb(c, s, sz=chg):
            bufa, _ = bufs(s, sz)
            _, _, _, _, semw = slots[s]
            pltpu.make_async_copy(bufa, out_hbm.at[pl.ds(base + c * chg, sz)],
                                  semw).wait()

        def step(c, s, *, wb_wait=True, fire1=True, next_sz=chg, sz=chg):
            wait_gather(c, s, sz)
            if fire1:
                if wb_wait:
                    wait_wb(c - 1, s ^ 1)
                fire_gather(c + 1, s ^ 1, next_sz)
            add_slot(s, sz)
            fire_wb(c, s, sz)

        fire_gather(0, 0)
        step(0, 0, wb_wait=False)

        def pair(r, cc):
            c = 2 * r + 1
            step(c, 1)
            step(c + 1, 0)
            return cc

        lax.fori_loop(0, (nfull - 2) // 2, pair, 0)
        step(nfull - 1, 1, next_sz=tail)
        step(nfull, 0, fire1=False, sz=tail)
        wait_wb(nfull - 1, 1)
        wait_wb(nfull, 0, tail)

    return k(a_tab, b_tab, src, dst)


def _edge_mlp(g, attr, w1c, g1, bt1, w2, b2, block_rows=2000):
    e, _ = g.shape
    de = attr.shape[1]

    def body(g_ref, a_ref, w1c_ref, g1_ref, bt1_ref, w2_ref, b2_ref, o_ref):
        x = g_ref[...] + _dot_t(a_ref[...], w1c_ref[...])
        mu = jnp.mean(x, axis=-1, keepdims=True)
        var = jnp.mean((x - mu) ** 2, axis=-1, keepdims=True)
        h = (x - mu) * lax.rsqrt(var + _EPS) * g1_ref[...] + bt1_ref[...]
        h = jnp.maximum(h, 0.0)
        o_ref[...] = _dot_t(h, w2_ref[...]) + b2_ref[...]

    return pl.pallas_call(
        body,
        grid=(e // block_rows,),
        in_specs=[
            pl.BlockSpec((block_rows, D), lambda i: (i, 0)),
            pl.BlockSpec((block_rows, de), lambda i: (i, 0)),
            pl.BlockSpec((D, de), lambda i: (0, 0)),
            pl.BlockSpec((1, D), lambda i: (0, 0)),
            pl.BlockSpec((1, D), lambda i: (0, 0)),
            pl.BlockSpec((D, D), lambda i: (0, 0)),
            pl.BlockSpec((1, D), lambda i: (0, 0)),
        ],
        out_specs=pl.BlockSpec((block_rows, D), lambda i: (i, 0)),
        out_shape=jax.ShapeDtypeStruct((e, D), jnp.float32),
    )(g, attr, w1c, g1.reshape(1, D), bt1.reshape(1, D), w2, b2.reshape(1, D))


def _sc_scatter(msg, dst, n):
    e_total = dst.shape[0]
    per_w = e_total // NW
    assert e_total % NW == 0 and per_w % 8 == 0
    chs = 80
    nfull = per_w // chs
    tail = per_w - nfull * chs
    if tail == 0:
        nfull -= 1
        tail = chs
    nsteps = nfull + 1
    assert nsteps >= 6 and tail % 8 == 0
    stripe = n // NS
    zr = 25
    nz = stripe // zr
    mesh = plsc.VectorSubcoreMesh(core_axis_name="c", subcore_axis_name="s",
                                  num_cores=NC, num_subcores=NS)

    @functools.partial(
        pl.kernel,
        out_type=jax.ShapeDtypeStruct((NC, n, ACC_W), jnp.float32),
        mesh=mesh,
        scratch_types=[
            pltpu.VMEM((per_w,), jnp.int32),
            pltpu.VMEM((chs, ACC_W), jnp.float32),
            pltpu.VMEM((chs, ACC_W), jnp.float32),
            pltpu.VMEM((zr, ACC_W), jnp.float32),
            pltpu.VMEM_SHARED((n, ACC_W), jnp.float32),
            pltpu.SemaphoreType.DMA,
            pltpu.SemaphoreType.DMA,
            pltpu.SemaphoreType.DMA,
            pltpu.SemaphoreType.DMA,
            pltpu.SemaphoreType.DMA,
        ],
        compiler_params=_SC_PARAMS,
    )
    def k(msg_hbm, dst_hbm, out_hbm, idx_all, sbuf0, sbuf1, zbuf, acc,
          seml0, seml1, semsc0, semsc1, semz):
        cid = lax.axis_index("c")
        sid = lax.axis_index("s")
        wid = sid * NC + cid
        base = wid * per_w
        slots = ((sbuf0, seml0, semsc0), (sbuf1, seml1, semsc1))

        pltpu.sync_copy(dst_hbm.at[pl.ds(base, per_w)], idx_all)

        zero16 = jnp.zeros((16,), jnp.float32)
        one_first = jnp.where(lax.iota(jnp.int32, 16) == 0, 1.0, 0.0)

        def zrow(i, cc):
            for j in range(ACC_W // 16):
                zbuf[i, pl.ds(j * 16, 16)] = zero16
            return cc

        lax.fori_loop(0, zr, zrow, 0)

        def srow(i, cc):
            sbuf0[i, pl.ds(D, 16)] = one_first
            sbuf1[i, pl.ds(D, 16)] = one_first
            return cc

        lax.fori_loop(0, chs, srow, 0)

        def zdst(j):
            return acc.at[pl.ds(sid * stripe + j * zr, zr)]

        for j in range(nz):
            pltpu.async_copy(zbuf, zdst(j), semz)
        for j in range(nz):
            pltpu.make_async_copy(zbuf, zdst(j), semz).wait()
        plsc.subcore_barrier()

        def sz_of(c):
            return tail if c == nsteps - 1 else chs

        def sb(s, sz):
            sbuf = slots[s][0]
            return sbuf if sz == chs else sbuf.at[pl.ds(0, sz)]

        def fire_load(c, s, sz=chs):
            _, seml, _ = slots[s]
            off = base + c * chs
            pltpu.async_copy(msg_hbm.at[pl.ds(off, sz)],
                             sb(s, sz).at[:, pl.ds(0, D)], seml)

        def wait_load(c, s, sz=chs):
            _, seml, _ = slots[s]
            off = base + c * chs
            pltpu.make_async_copy(msg_hbm.at[pl.ds(off, sz)],
                                  sb(s, sz).at[:, pl.ds(0, D)], seml).wait()

        def fire_scatter(c, s, sz=chs):
            _, _, semsc = slots[s]
            idxv = idx_all.at[pl.ds(c * chs, sz)]
            pltpu.async_copy(sb(s, sz), acc.at[idxv], semsc, add=True)

        def wait_scatter(c, s, sz=chs):
            _, _, semsc = slots[s]
            idxv = idx_all.at[pl.ds(c * chs, sz)]
            pltpu.make_async_copy(sb(s, sz), acc.at[idxv], semsc).wait()

        def step(c, s, *, scat_wait=True, fire1=True, sz=chs, next_sz=chs):
            wait_load(c, s, sz)
            if fire1:
                if scat_wait:
                    wait_scatter(c - 1, s ^ 1)
                fire_load(c + 1, s ^ 1, next_sz)
            fire_scatter(c, s, sz)

        fire_load(0, 0)
        step(0, 0, scat_wait=False)
        pe = 1
        if (nsteps - 3) % 2 == 1:
            step(1, 1)
            pe = 2
        c0 = pe

        def pair(r, cc):
            c = c0 + 2 * r
            step(c, c0 % 2)
            step(c + 1, (c0 % 2) ^ 1)
            return cc

        lax.fori_loop(0, (nsteps - 2 - pe) // 2, pair, 0)
        step(nsteps - 2, (nsteps - 2) % 2, next_sz=tail)
        step(nsteps - 1, (nsteps - 1) % 2, fire1=False, sz=tail)
        wait_scatter(nsteps - 2, (nsteps - 2) % 2)
        wait_scatter(nsteps - 1, (nsteps - 1) % 2, tail)
        plsc.subcore_barrier()

        def ddst(j):
            r0 = sid * stripe + j * zr
            return (acc.at[pl.ds(r0, zr)], out_hbm.at[cid, pl.ds(r0, zr)])

        for j in range(nz):
            s_, d_ = ddst(j)
            pltpu.async_copy(s_, d_, semz)
        for j in range(nz):
            s_, d_ = ddst(j)
            pltpu.make_async_copy(s_, d_, semz).wait()

    return k(msg, dst)


def _gru_ln(psums, h_old, wih, whh, bih, bhh, g, b, w1a_next=None,
            block_rows=1000):
    n = h_old.shape[0]
    np_in = len(psums)

    def body(*refs):
        p_refs = refs[:np_in]
        h_ref, wih_ref, whh_ref, bih_ref, bhh_ref, g_ref, b_ref = \
            refs[np_in:np_in + 7]
        rest = refs[np_in + 7:]
        parts = [p_ref[i] for p_ref in p_refs for i in range(NC)]
        s = parts[0][:, :D]
        c = parts[0][:, D:D + 1]
        for p in parts[1:]:
            s = s + p[:, :D]
            c = c + p[:, D:D + 1]
        agg = s / jnp.maximum(c, 1.0)
        h = h_ref[...]
        gi = _dot_t(agg, wih_ref[...]) + bih_ref[...]
        gh = _dot_t(h, whh_ref[...]) + bhh_ref[...]
        r = jax.nn.sigmoid(gi[:, :D] + gh[:, :D])
        z = jax.nn.sigmoid(gi[:, D:2 * D] + gh[:, D:2 * D])
        nn = jnp.tanh(gi[:, 2 * D:] + r * gh[:, 2 * D:])
        hn = (1.0 - z) * nn + z * h
        mu = jnp.mean(hn, axis=-1, keepdims=True)
        var = jnp.mean((hn - mu) ** 2, axis=-1, keepdims=True)
        out = (hn - mu) * lax.rsqrt(var + _EPS) * g_ref[...] + b_ref[...]
        if w1a_next is None:
            rest[-1][...] = out
        else:
            wa_ref, o_ref, a_ref = rest
            o_ref[...] = out
            a_ref[...] = _dot_t(out, wa_ref[...])

    rows = lambda i: (i, 0)
    full = lambda i: (0, 0)
    in_specs = [
        pl.BlockSpec((NC, block_rows, ACC_W), lambda i: (0, i, 0))
        for _ in range(np_in)
    ] + [
        pl.BlockSpec((block_rows, D), rows),
        pl.BlockSpec((3 * D, D), full),
        pl.BlockSpec((3 * D, D), full),
        pl.BlockSpec((1, 3 * D), full),
        pl.BlockSpec((1, 3 * D), full),
        pl.BlockSpec((1, D), full),
        pl.BlockSpec((1, D), full),
    ]
    args = list(psums) + [h_old, wih, whh, bih.reshape(1, 3 * D),
            bhh.reshape(1, 3 * D), g.reshape(1, D), b.reshape(1, D)]
    out_spec = pl.BlockSpec((block_rows, D), rows)
    out_ty = jax.ShapeDtypeStruct((n, D), jnp.float32)
    if w1a_next is None:
        out_specs, out_shape = out_spec, out_ty
    else:
        in_specs.append(pl.BlockSpec((D, D), full))
        args.append(w1a_next)
        out_specs, out_shape = [out_spec, out_spec], [out_ty, out_ty]
    return pl.pallas_call(
        body,
        grid=(n // block_rows,),
        in_specs=in_specs,
        out_specs=out_specs,
        out_shape=out_shape,
    )(*args)


def kernel(h_nodes, h_nets, edge_index_m2n, edge_attr_m2n, edge_index_n2m,
           edge_attr_n2m, m2n_W1, m2n_b1, m2n_g1, m2n_bt1, m2n_W2, m2n_b2,
           n2m_W1, n2m_b1, n2m_g1, n2m_bt1, n2m_W2, n2m_b2,
           gru_net_Wih, gru_net_Whh, gru_net_bih, gru_net_bhh,
           gru_mac_Wih, gru_mac_Whh, gru_mac_bih, gru_mac_bhh,
           ln_net_g, ln_net_b, ln_mac_g, ln_mac_b):
    src_m = edge_index_m2n[0].astype(jnp.int32)
    dst_n = edge_index_m2n[1].astype(jnp.int32)
    src_n = edge_index_n2m[0].astype(jnp.int32)
    dst_m = edge_index_n2m[1].astype(jnp.int32)
    n_nodes = h_nodes.shape[0]
    n_nets = h_nets.shape[0]

    e_total = src_m.shape[0]
    ns = 2
    es = e_total // ns
    sl = [slice(i * es, (i + 1) * es) for i in range(ns)]

    a1, b1t, b2t = _tables_phase1(h_nodes, h_nets, m2n_W1[:, :D],
                                  m2n_W1[:, D:2 * D], m2n_b1,
                                  n2m_W1[:, D:2 * D], n2m_b1)
    g1 = [_sc_gather_add(a1, b1t, src_m[s], dst_n[s]) for s in sl]
    msg1 = [_edge_mlp(g, edge_attr_m2n[s], m2n_W1[:, 2 * D:], m2n_g1,
                      m2n_bt1, m2n_W2, m2n_b2) for g, s in zip(g1, sl)]
    psum1 = [_sc_scatter(m, dst_n[s], n_nets) for m, s in zip(msg1, sl)]
    h_nets_new, a2 = _gru_ln(tuple(psum1), h_nets, gru_net_Wih, gru_net_Whh,
                             gru_net_bih, gru_net_bhh, ln_net_g, ln_net_b,
                             w1a_next=n2m_W1[:, :D])

    g2 = [_sc_gather_add(a2, b2t, src_n[s], dst_m[s]) for s in sl]
    msg2 = [_edge_mlp(g, edge_attr_n2m[s], n2m_W1[:, 2 * D:], n2m_g1,
                      n2m_bt1, n2m_W2, n2m_b2) for g, s in zip(g2, sl)]
    psum2 = [_sc_scatter(m, dst_m[s], n_nodes) for m, s in zip(msg2, sl)]
    h_nodes_new = _gru_ln(tuple(psum2), h_nodes, gru_mac_Wih, gru_mac_Whh,
                          gru_mac_bih, gru_mac_bhh, ln_mac_g, ln_mac_b)
    return (h_nodes_new, h_nets_new)


# speedup vs baseline: 1.0777x; 1.0040x over previous
import functools

import jax
import jax.numpy as jnp
from jax import lax
from jax.experimental import pallas as pl
from jax.experimental.pallas import tpu as pltpu
from jax.experimental.pallas import tpu_sc as plsc

D = 128
ACC_W = D + 8
NC = 2
NS = 16
NW = NC * NS
CH = 80
_EPS = 1e-5
_PREC = jax.lax.Precision.DEFAULT
_SC_PARAMS = pltpu.CompilerParams(use_tc_tiling_on_sc=False)


def _dot_t(x, w):
    return lax.dot_general(x, w, (((1,), (1,)), ((), ())),
                           preferred_element_type=jnp.float32, precision=_PREC)


def _tables_phase1(h_nodes, h_nets, w1a1, w1b1, b11, w1b2, b12,
                   block_rows=1000):
    n = h_nodes.shape[0]

    def body(hn_ref, hv_ref, wa_ref, wb1_ref, b11_ref, wb2_ref, b12_ref,
             a1_ref, b1_ref, b2_ref):
        hn = hn_ref[...]
        a1_ref[...] = _dot_t(hn, wa_ref[...])
        b1_ref[...] = _dot_t(hv_ref[...], wb1_ref[...]) + b11_ref[...]
        b2_ref[...] = _dot_t(hn, wb2_ref[...]) + b12_ref[...]

    rows = lambda i: (i, 0)
    full = lambda i: (0, 0)
    out = jax.ShapeDtypeStruct((n, D), jnp.float32)
    return pl.pallas_call(
        body,
        grid=(n // block_rows,),
        in_specs=[
            pl.BlockSpec((block_rows, D), rows),
            pl.BlockSpec((block_rows, D), rows),
            pl.BlockSpec((D, D), full),
            pl.BlockSpec((D, D), full),
            pl.BlockSpec((1, D), full),
            pl.BlockSpec((D, D), full),
            pl.BlockSpec((1, D), full),
        ],
        out_specs=[pl.BlockSpec((block_rows, D), rows)] * 3,
        out_shape=[out, out, out],
    )(h_nodes, h_nets, w1a1, w1b1, b11.reshape(1, D), w1b2, b12.reshape(1, D))


def _sc_gather_add(a_tab, b_tab, src, dst):
    e_total = src.shape[0]
    per_w = e_total // NW
    assert e_total % NW == 0 and per_w % 8 == 0
    for chg in range(128, 7, -8):
        nfull = per_w // chg
        tail = per_w - nfull * chg
        if nfull >= 6 and nfull % 2 == 0 and tail > 0:
            break
    assert nfull % 2 == 0 and nfull >= 6 and 0 < tail <= chg and tail % 8 == 0
    mesh = plsc.VectorSubcoreMesh(core_axis_name="c", subcore_axis_name="s",
                                  num_cores=NC, num_subcores=NS)

    @functools.partial(
        pl.kernel,
        out_type=jax.ShapeDtypeStruct((e_total, D), jnp.float32),
        mesh=mesh,
        scratch_types=[
            pltpu.VMEM((per_w,), jnp.int32),
            pltpu.VMEM((per_w,), jnp.int32),
            pltpu.VMEM((chg, D), jnp.float32),
            pltpu.VMEM((chg, D), jnp.float32),
            pltpu.VMEM((chg, D), jnp.float32),
            pltpu.VMEM((chg, D), jnp.float32),
            pltpu.SemaphoreType.DMA,
            pltpu.SemaphoreType.DMA,
            pltpu.SemaphoreType.DMA,
            pltpu.SemaphoreType.DMA,
            pltpu.SemaphoreType.DMA,
            pltpu.SemaphoreType.DMA,
        ],
        compiler_params=_SC_PARAMS,
    )
    def k(a_hbm, b_hbm, src_hbm, dst_hbm, out_hbm,
          sidx_all, didx_all, bufa0, bufa1, bufb0, bufb1,
          sema0, sema1, semb0, semb1, semw0, semw1):
        wid = lax.axis_index("s") * NC + lax.axis_index("c")
        base = wid * per_w
        slots = ((bufa0, bufb0, sema0, semb0, semw0),
                 (bufa1, bufb1, sema1, semb1, semw1))

        pltpu.sync_copy(src_hbm.at[pl.ds(base, per_w)], sidx_all)
        pltpu.sync_copy(dst_hbm.at[pl.ds(base, per_w)], didx_all)

        def bufs(s, sz):
            bufa, bufb, _, _, _ = slots[s]
            if sz == chg:
                return bufa, bufb
            return bufa.at[pl.ds(0, sz)], bufb.at[pl.ds(0, sz)]

        def fire_gather(c, s, sz=chg):
            bufa, bufb = bufs(s, sz)
            _, _, sema, semb, _ = slots[s]
            off = c * chg
            pltpu.async_copy(a_hbm.at[sidx_all.at[pl.ds(off, sz)]], bufa, sema)
            pltpu.async_copy(b_hbm.at[didx_all.at[pl.ds(off, sz)]], bufb, semb)

        def wait_gather(c, s, sz=chg):
            bufa, bufb = bufs(s, sz)
            _, _, sema, semb, _ = slots[s]
            off = c * chg
            pltpu.make_async_copy(a_hbm.at[sidx_all.at[pl.ds(off, sz)]], bufa,
                                  sema).wait()
            pltpu.make_async_copy(b_hbm.at[didx_all.at[pl.ds(off, sz)]], bufb,
                                  semb).wait()

        def add_slot(s, sz=chg):
            bufa, bufb, _, _, _ = slots[s]

            def row(i, cc):
                for j in range(D // 16):
                    sl = pl.ds(j * 16, 16)
                    plsc.addupdate(bufa.at[i, sl], bufb[i, sl])
                return cc

            lax.fori_loop(0, sz, row, 0)

        def fire_wb(c, s, sz=chg):
            bufa, _ = bufs(s, sz)
            _, _, _, _, semw = slots[s]
            pltpu.async_copy(bufa, out_hbm.at[pl.ds(base + c * chg, sz)], semw)

        def wait_wb(c, s, sz=chg):
            bufa, _ = bufs(s, sz)
            _, _, _, _, semw = slots[s]
            pltpu.make_async_copy(bufa, out_hbm.at[pl.ds(base + c * chg, sz)],
                                  semw).wait()

        def step(c, s, *, wb_wait=True, fire1=True, next_sz=chg, sz=chg):
            wait_gather(c, s, sz)
            if fire1:
                if wb_wait:
                    wait_wb(c - 1, s ^ 1)
                fire_gather(c + 1, s ^ 1, next_sz)
            add_slot(s, sz)
            fire_wb(c, s, sz)

        fire_gather(0, 0)
        step(0, 0, wb_wait=False)

        def pair(r, cc):
            c = 2 * r + 1
            step(c, 1)
            step(c + 1, 0)
            return cc

        lax.fori_loop(0, (nfull - 2) // 2, pair, 0)
        step(nfull - 1, 1, next_sz=tail)
        step(nfull, 0, fire1=False, sz=tail)
        wait_wb(nfull - 1, 1)
        wait_wb(nfull, 0, tail)

    return k(a_tab, b_tab, src, dst)


def _edge_mlp(g, attr, w1c, g1, bt1, w2, b2, block_rows=2000):
    e, _ = g.shape
    de = attr.shape[1]

    def body(g_ref, a_ref, w1c_ref, g1_ref, bt1_ref, w2_ref, b2_ref, o_ref):
        x = g_ref[...] + _dot_t(a_ref[...], w1c_ref[...])
        mu = jnp.mean(x, axis=-1, keepdims=True)
        var = jnp.mean((x - mu) ** 2, axis=-1, keepdims=True)
        h = (x - mu) * lax.rsqrt(var + _EPS) * g1_ref[...] + bt1_ref[...]
        h = jnp.maximum(h, 0.0)
        o_ref[...] = _dot_t(h, w2_ref[...]) + b2_ref[...]

    return pl.pallas_call(
        body,
        grid=(e // block_rows,),
        in_specs=[
            pl.BlockSpec((block_rows, D), lambda i: (i, 0)),
            pl.BlockSpec((block_rows, de), lambda i: (i, 0)),
            pl.BlockSpec((D, de), lambda i: (0, 0)),
            pl.BlockSpec((1, D), lambda i: (0, 0)),
            pl.BlockSpec((1, D), lambda i: (0, 0)),
            pl.BlockSpec((D, D), lambda i: (0, 0)),
            pl.BlockSpec((1, D), lambda i: (0, 0)),
        ],
        out_specs=pl.BlockSpec((block_rows, D), lambda i: (i, 0)),
        out_shape=jax.ShapeDtypeStruct((e, D), jnp.float32),
    )(g, attr, w1c, g1.reshape(1, D), bt1.reshape(1, D), w2, b2.reshape(1, D))


def _sc_scatter(msg, dst, n):
    e_total = dst.shape[0]
    per_w = e_total // NW
    assert e_total % NW == 0 and per_w % 8 == 0
    chs = 80
    nfull = per_w // chs
    tail = per_w - nfull * chs
    if tail == 0:
        nfull -= 1
        tail = chs
    nsteps = nfull + 1
    assert nsteps >= 6 and tail % 8 == 0
    stripe = n // NS
    zr = 25
    nz = stripe // zr
    mesh = plsc.VectorSubcoreMesh(core_axis_name="c", subcore_axis_name="s",
                                  num_cores=NC, num_subcores=NS)

    @functools.partial(
        pl.kernel,
        out_type=jax.ShapeDtypeStruct((NC, n, ACC_W), jnp.float32),
        mesh=mesh,
        scratch_types=[
            pltpu.VMEM((per_w,), jnp.int32),
            pltpu.VMEM((chs, ACC_W), jnp.float32),
            pltpu.VMEM((chs, ACC_W), jnp.float32),
            pltpu.VMEM((zr, ACC_W), jnp.float32),
            pltpu.VMEM_SHARED((n, ACC_W), jnp.float32),
            pltpu.SemaphoreType.DMA,
            pltpu.SemaphoreType.DMA,
            pltpu.SemaphoreType.DMA,
            pltpu.SemaphoreType.DMA,
            pltpu.SemaphoreType.DMA,
        ],
        compiler_params=_SC_PARAMS,
    )
    def k(msg_hbm, dst_hbm, out_hbm, idx_all, sbuf0, sbuf1, zbuf, acc,
          seml0, seml1, semsc0, semsc1, semz):
        cid = lax.axis_index("c")
        sid = lax.axis_index("s")
        wid = sid * NC + cid
        base = wid * per_w
        slots = ((sbuf0, seml0, semsc0), (sbuf1, seml1, semsc1))

        pltpu.sync_copy(dst_hbm.at[pl.ds(base, per_w)], idx_all)

        zero16 = jnp.zeros((16,), jnp.float32)
        cnt_off = ACC_W - 16
        one_cnt = jnp.where(lax.iota(jnp.int32, 16) == D - cnt_off, 1.0, 0.0)

        def zrow(i, cc):
            for j in range(D // 16):
                zbuf[i, pl.ds(j * 16, 16)] = zero16
            zbuf[i, pl.ds(cnt_off, 16)] = zero16
            return cc

        lax.fori_loop(0, zr, zrow, 0)

        def srow(i, cc):
            sbuf0[i, pl.ds(cnt_off, 16)] = one_cnt
            sbuf1[i, pl.ds(cnt_off, 16)] = one_cnt
            return cc

        lax.fori_loop(0, chs, srow, 0)

        def zdst(j):
            return acc.at[pl.ds(sid * stripe + j * zr, zr)]

        for j in range(nz):
            pltpu.async_copy(zbuf, zdst(j), semz)
        for j in range(nz):
            pltpu.make_async_copy(zbuf, zdst(j), semz).wait()
        plsc.subcore_barrier()

        def sz_of(c):
            return tail if c == nsteps - 1 else chs

        def sb(s, sz):
            sbuf = slots[s][0]
            return sbuf if sz == chs else sbuf.at[pl.ds(0, sz)]

        def fire_load(c, s, sz=chs):
            _, seml, _ = slots[s]
            off = base + c * chs
            pltpu.async_copy(msg_hbm.at[pl.ds(off, sz)],
                             sb(s, sz).at[:, pl.ds(0, D)], seml)

        def wait_load(c, s, sz=chs):
            _, seml, _ = slots[s]
            off = base + c * chs
            pltpu.make_async_copy(msg_hbm.at[pl.ds(off, sz)],
                                  sb(s, sz).at[:, pl.ds(0, D)], seml).wait()

        def fire_scatter(c, s, sz=chs):
            _, _, semsc = slots[s]
            idxv = idx_all.at[pl.ds(c * chs, sz)]
            pltpu.async_copy(sb(s, sz), acc.at[idxv], semsc, add=True)

        def wait_scatter(c, s, sz=chs):
            _, _, semsc = slots[s]
            idxv = idx_all.at[pl.ds(c * chs, sz)]
            pltpu.make_async_copy(sb(s, sz), acc.at[idxv], semsc).wait()

        def step(c, s, *, scat_wait=True, fire1=True, sz=chs, next_sz=chs):
            wait_load(c, s, sz)
            if fire1:
                if scat_wait:
                    wait_scatter(c - 1, s ^ 1)
                fire_load(c + 1, s ^ 1, next_sz)
            fire_scatter(c, s, sz)

        fire_load(0, 0)
        step(0, 0, scat_wait=False)
        pe = 1
        if (nsteps - 3) % 2 == 1:
            step(1, 1)
            pe = 2
        c0 = pe

        def pair(r, cc):
            c = c0 + 2 * r
            step(c, c0 % 2)
            step(c + 1, (c0 % 2) ^ 1)
            return cc

        lax.fori_loop(0, (nsteps - 2 - pe) // 2, pair, 0)
        step(nsteps - 2, (nsteps - 2) % 2, next_sz=tail)
        step(nsteps - 1, (nsteps - 1) % 2, fire1=False, sz=tail)
        wait_scatter(nsteps - 2, (nsteps - 2) % 2)
        wait_scatter(nsteps - 1, (nsteps - 1) % 2, tail)
        plsc.subcore_barrier()

        def ddst(j):
            r0 = sid * stripe + j * zr
            return (acc.at[pl.ds(r0, zr)], out_hbm.at[cid, pl.ds(r0, zr)])

        for j in range(nz):
            s_, d_ = ddst(j)
            pltpu.async_copy(s_, d_, semz)
        for j in range(nz):
            s_, d_ = ddst(j)
            pltpu.make_async_copy(s_, d_, semz).wait()

    return k(msg, dst)


def _gru_ln(psums, h_old, wih, whh, bih, bhh, g, b, w1a_next=None,
            block_rows=1000):
    n = h_old.shape[0]
    np_in = len(psums)

    def body(*refs):
        p_refs = refs[:np_in]
        h_ref, wih_ref, whh_ref, bih_ref, bhh_ref, g_ref, b_ref = \
            refs[np_in:np_in + 7]
        rest = refs[np_in + 7:]
        parts = [p_ref[i] for p_ref in p_refs for i in range(NC)]
        s = parts[0][:, :D]
        c = parts[0][:, D:D + 1]
        for p in parts[1:]:
            s = s + p[:, :D]
            c = c + p[:, D:D + 1]
        agg = s / jnp.maximum(c, 1.0)
        h = h_ref[...]
        gi = _dot_t(agg, wih_ref[...]) + bih_ref[...]
        gh = _dot_t(h, whh_ref[...]) + bhh_ref[...]
        r = jax.nn.sigmoid(gi[:, :D] + gh[:, :D])
        z = jax.nn.sigmoid(gi[:, D:2 * D] + gh[:, D:2 * D])
        nn = jnp.tanh(gi[:, 2 * D:] + r * gh[:, 2 * D:])
        hn = (1.0 - z) * nn + z * h
        mu = jnp.mean(hn, axis=-1, keepdims=True)
        var = jnp.mean((hn - mu) ** 2, axis=-1, keepdims=True)
        out = (hn - mu) * lax.rsqrt(var + _EPS) * g_ref[...] + b_ref[...]
        if w1a_next is None:
            rest[-1][...] = out
        else:
            wa_ref, o_ref, a_ref = rest
            o_ref[...] = out
            a_ref[...] = _dot_t(out, wa_ref[...])

    rows = lambda i: (i, 0)
    full = lambda i: (0, 0)
    in_specs = [
        pl.BlockSpec((NC, block_rows, ACC_W), lambda i: (0, i, 0))
        for _ in range(np_in)
    ] + [
        pl.BlockSpec((block_rows, D), rows),
        pl.BlockSpec((3 * D, D), full),
        pl.BlockSpec((3 * D, D), full),
        pl.BlockSpec((1, 3 * D), full),
        pl.BlockSpec((1, 3 * D), full),
        pl.BlockSpec((1, D), full),
        pl.BlockSpec((1, D), full),
    ]
    args = list(psums) + [h_old, wih, whh, bih.reshape(1, 3 * D),
            bhh.reshape(1, 3 * D), g.reshape(1, D), b.reshape(1, D)]
    out_spec = pl.BlockSpec((block_rows, D), rows)
    out_ty = jax.ShapeDtypeStruct((n, D), jnp.float32)
    if w1a_next is None:
        out_specs, out_shape = out_spec, out_ty
    else:
        in_specs.append(pl.BlockSpec((D, D), full))
        args.append(w1a_next)
        out_specs, out_shape = [out_spec, out_spec], [out_ty, out_ty]
    return pl.pallas_call(
        body,
        grid=(n // block_rows,),
        in_specs=in_specs,
        out_specs=out_specs,
        out_shape=out_shape,
    )(*args)


def kernel(h_nodes, h_nets, edge_index_m2n, edge_attr_m2n, edge_index_n2m,
           edge_attr_n2m, m2n_W1, m2n_b1, m2n_g1, m2n_bt1, m2n_W2, m2n_b2,
           n2m_W1, n2m_b1, n2m_g1, n2m_bt1, n2m_W2, n2m_b2,
           gru_net_Wih, gru_net_Whh, gru_net_bih, gru_net_bhh,
           gru_mac_Wih, gru_mac_Whh, gru_mac_bih, gru_mac_bhh,
           ln_net_g, ln_net_b, ln_mac_g, ln_mac_b):
    src_m = edge_index_m2n[0].astype(jnp.int32)
    dst_n = edge_index_m2n[1].astype(jnp.int32)
    src_n = edge_index_n2m[0].astype(jnp.int32)
    dst_m = edge_index_n2m[1].astype(jnp.int32)
    n_nodes = h_nodes.shape[0]
    n_nets = h_nets.shape[0]

    e_total = src_m.shape[0]
    cut = (2 * e_total // 5 // (8 * NW)) * (8 * NW)
    sl = [slice(0, cut), slice(cut, e_total)]

    a1, b1t, b2t = _tables_phase1(h_nodes, h_nets, m2n_W1[:, :D],
                                  m2n_W1[:, D:2 * D], m2n_b1,
                                  n2m_W1[:, D:2 * D], n2m_b1)
    g1 = [_sc_gather_add(a1, b1t, src_m[s], dst_n[s]) for s in sl]
    msg1 = [_edge_mlp(g, edge_attr_m2n[s], m2n_W1[:, 2 * D:], m2n_g1,
                      m2n_bt1, m2n_W2, m2n_b2) for g, s in zip(g1, sl)]
    psum1 = [_sc_scatter(m, dst_n[s], n_nets) for m, s in zip(msg1, sl)]
    h_nets_new, a2 = _gru_ln(tuple(psum1), h_nets, gru_net_Wih, gru_net_Whh,
                             gru_net_bih, gru_net_bhh, ln_net_g, ln_net_b,
                             w1a_next=n2m_W1[:, :D])

    g2 = [_sc_gather_add(a2, b2t, src_n[s], dst_m[s]) for s in sl]
    msg2 = [_edge_mlp(g, edge_attr_n2m[s], n2m_W1[:, 2 * D:], n2m_g1,
                      n2m_bt1, n2m_W2, n2m_b2) for g, s in zip(g2, sl)]
    psum2 = [_sc_scatter(m, dst_m[s], n_nodes) for m, s in zip(msg2, sl)]
    h_nodes_new = _gru_ln(tuple(psum2), h_nodes, gru_mac_Wih, gru_mac_Whh,
                          gru_mac_bih, gru_mac_bhh, ln_mac_g, ln_mac_b)
    return (h_nodes_new, h_nets_new)
